# Initial kernel scaffold; baseline (speedup 1.0000x reference)
import jax
import jax.numpy as jnp
from jax.experimental import pallas as pl


def kernel(x, edge_index, h_W, h_b, gat_W, att_src, att_dst, res_W, res_b, ln_g, ln_b, betas):
    raise NotImplementedError("write your pallas kernel here")


# speedup vs baseline: 53.3134x; 53.3134x over previous
import functools

import jax
import jax.numpy as jnp
import numpy as np
from jax import lax
from jax.experimental import pallas as pl
from jax.experimental.pallas import tpu as pltpu
from jax.experimental.pallas import tpu_sc as plsc

N = 10000
D = 128
H = 8
C = 16
E = 320000

_NC = 2
_NS = 16
_NW = _NC * _NS
CK = 128
CPT = 79
EPT = CPT * CK
E_PAD = _NW * EPT
NROW = N + 112
RPT = NROW // _NS
FW = 144

_SEL = np.zeros((D, H), np.float32)
_SEL[np.arange(D), np.arange(D) // C] = 1.0

_B = 1000


def _sc_edge_body(srcp, dstp, xpa, ald16, out, acc_sh, msg_v, ald_v,
                  src_v, dst_v, dst2_v, sem):
    cid = lax.axis_index("c")
    sid = lax.axis_index("s")
    wid = cid * _NS + sid

    def _zero_msg(i, carry):
        for j in range(FW // 16):
            msg_v[i, pl.ds(j * 16, 16)] = jnp.zeros((16,), jnp.float32)
        return carry

    lax.fori_loop(0, CK, _zero_msg, 0)
    zbase = sid * RPT
    for j in range(4):
        pltpu.sync_copy(msg_v, acc_sh.at[pl.ds(zbase + j * CK, CK)])
    pltpu.sync_copy(msg_v.at[pl.ds(0, RPT - 4 * CK)],
                    acc_sh.at[pl.ds(zbase + 4 * CK, RPT - 4 * CK)])
    plsc.subcore_barrier()

    def _chunk(c, carry):
        base = wid * EPT + c * CK
        pltpu.sync_copy(srcp.at[pl.ds(base, CK)], src_v)
        pltpu.sync_copy(dstp.at[pl.ds(base, CK)], dst_v)
        for j in range(CK // 16):
            dst2_v[pl.ds(j * 16, 16)] = jnp.minimum(
                dst_v[pl.ds(j * 16, 16)], N - 1)
        pltpu.async_copy(xpa.at[src_v], msg_v, sem).wait()
        pltpu.async_copy(ald16.at[dst2_v], ald_v, sem).wait()

        def _edge(e, ecarry):
            al = msg_v[e, pl.ds(D, 16)] + ald_v[e, pl.ds(0, 16)]
            exv = jnp.exp(jnp.maximum(al, 0.2 * al))
            msg_v[e, pl.ds(D, 16)] = exv
            for h in range(H):
                w = jnp.full((16,), exv[h], jnp.float32)
                msg_v[e, pl.ds(h * 16, 16)] = msg_v[e, pl.ds(h * 16, 16)] * w
            return ecarry

        lax.fori_loop(0, CK, _edge, 0)
        pltpu.sync_copy(msg_v, acc_sh.at[dst_v], add=True)
        return carry

    lax.fori_loop(0, CPT, _chunk, 0)
    plsc.subcore_barrier()

    rbase = sid * RPT
    obase = cid * NROW + rbase
    for j in range(4):
        pltpu.sync_copy(acc_sh.at[pl.ds(rbase + j * CK, CK)], msg_v)
        pltpu.sync_copy(msg_v, out.at[pl.ds(obase + j * CK, CK)])
    rem = RPT - 4 * CK
    pltpu.sync_copy(acc_sh.at[pl.ds(rbase + 4 * CK, rem)],
                    msg_v.at[pl.ds(0, rem)])
    pltpu.sync_copy(msg_v.at[pl.ds(0, rem)], out.at[pl.ds(obase + 4 * CK, rem)])


@functools.cache
def _make_sc_edge():
    return pl.kernel(
        _sc_edge_body,
        out_type=jax.ShapeDtypeStruct((_NC * NROW, FW), jnp.float32),
        mesh=plsc.VectorSubcoreMesh(core_axis_name="c", subcore_axis_name="s",
                                    num_cores=_NC, num_subcores=_NS),
        scratch_types=[
            pltpu.VMEM_SHARED((NROW, FW), jnp.float32),
            pltpu.VMEM((CK, FW), jnp.float32),
            pltpu.VMEM((CK, 16), jnp.float32),
            pltpu.VMEM((CK,), jnp.int32),
            pltpu.VMEM((CK,), jnp.int32),
            pltpu.VMEM((CK,), jnp.int32),
            pltpu.SemaphoreType.DMA,
        ],
        compiler_params=pltpu.CompilerParams(use_tc_tiling_on_sc=False),
    )


def _sc_edge(*args):
    return _make_sc_edge()(*args)


def _pre_math(xp, asrc, adst, sel):
    als = jnp.dot(xp * asrc, sel, preferred_element_type=jnp.float32)
    ald = jnp.dot(xp * adst, sel, preferred_element_type=jnp.float32)
    z = jnp.zeros((xp.shape[0], H), jnp.float32)
    return (jnp.concatenate([xp, als, z], axis=1),
            jnp.concatenate([ald, z], axis=1))


def _tc_pre_body(x_ref, w_ref, asrc_ref, adst_ref, sel_ref, xpa_ref, ald_ref):
    xp = jnp.dot(x_ref[...], w_ref[...], preferred_element_type=jnp.float32)
    xpa, ald = _pre_math(xp, asrc_ref[...], adst_ref[...], sel_ref[...])
    xpa_ref[...] = xpa
    ald_ref[...] = ald


def _post_math(acc_ref, x, hw, hb, rw, rb, g, b, beta, selT):
    S = acc_ref[0] + acc_ref[1]
    den128 = jnp.dot(S[:, D:D + H], selT, preferred_element_type=jnp.float32)
    gat = S[:, :D] / (den128 + 1e-16)
    h = jnp.maximum(jnp.dot(x, hw, preferred_element_type=jnp.float32) + hb, 0.0)
    res = jnp.dot(x, rw, preferred_element_type=jnp.float32) + rb
    av = jnp.maximum(gat + res, 0.0)
    t = av * h
    m = jnp.mean(t, axis=-1, keepdims=True)
    var = jnp.mean((t - m) ** 2, axis=-1, keepdims=True)
    ln = (t - m) * lax.rsqrt(var + 1e-5) * g + b
    return (1.0 - beta) * ln + beta * av


def _tc_mid_body(acc_ref, x_ref, hw_ref, hb_ref, rw_ref, rb_ref, g_ref, b_ref,
                 beta_ref, selT_ref, w2_ref, asrc_ref, adst_ref, sel_ref,
                 xn_ref, xpa_ref, ald_ref):
    xn = _post_math(acc_ref, x_ref[...], hw_ref[...], hb_ref[...], rw_ref[...],
                    rb_ref[...], g_ref[...], b_ref[...], beta_ref[...],
                    selT_ref[...])
    xn_ref[...] = xn
    xp = jnp.dot(xn, w2_ref[...], preferred_element_type=jnp.float32)
    xpa, ald = _pre_math(xp, asrc_ref[...], adst_ref[...], sel_ref[...])
    xpa_ref[...] = xpa
    ald_ref[...] = ald


def _tc_final_body(acc_ref, x_ref, hw_ref, hb_ref, rw_ref, rb_ref, g_ref,
                   b_ref, beta_ref, selT_ref, out_ref):
    xn = _post_math(acc_ref, x_ref[...], hw_ref[...], hb_ref[...], rw_ref[...],
                    rb_ref[...], g_ref[...], b_ref[...], beta_ref[...],
                    selT_ref[...])
    out_ref[...] = xn + x_ref[...]


def _row_spec(w):
    return pl.BlockSpec((_B, w), lambda i: (i, 0))


def _full_spec(r, c):
    return pl.BlockSpec((r, c), lambda i: (0, 0))


_ACC_SPEC = pl.BlockSpec((_NC, _B, FW), lambda i: (0, i, 0))

_tc_pre = pl.pallas_call(
    _tc_pre_body,
    grid=(N // _B,),
    in_specs=[_row_spec(D), _full_spec(D, D), _full_spec(1, D),
              _full_spec(1, D), _full_spec(D, H)],
    out_specs=[_row_spec(FW), _row_spec(16)],
    out_shape=[jax.ShapeDtypeStruct((N, FW), jnp.float32),
               jax.ShapeDtypeStruct((N, 16), jnp.float32)],
)

_POST_SPECS = [_ACC_SPEC, _row_spec(D), _full_spec(D, D), _full_spec(1, D),
               _full_spec(D, D), _full_spec(1, D), _full_spec(1, D),
               _full_spec(1, D), _full_spec(1, D), _full_spec(H, D)]

_tc_mid = pl.pallas_call(
    _tc_mid_body,
    grid=(N // _B,),
    in_specs=_POST_SPECS + [_full_spec(D, D), _full_spec(1, D),
                            _full_spec(1, D), _full_spec(D, H)],
    out_specs=[_row_spec(D), _row_spec(FW), _row_spec(16)],
    out_shape=[jax.ShapeDtypeStruct((N, D), jnp.float32),
               jax.ShapeDtypeStruct((N, FW), jnp.float32),
               jax.ShapeDtypeStruct((N, 16), jnp.float32)],
)

_tc_final = pl.pallas_call(
    _tc_final_body,
    grid=(N // _B,),
    in_specs=_POST_SPECS,
    out_specs=_row_spec(D),
    out_shape=jax.ShapeDtypeStruct((N, D), jnp.float32),
)


def kernel(x, edge_index, h_W, h_b, gat_W, att_src, att_dst, res_W, res_b,
           ln_g, ln_b, betas):
    pad = E_PAD - E
    srcp = jnp.concatenate([edge_index[0], jnp.zeros((pad,), jnp.int32)])
    dstp = jnp.concatenate([edge_index[1], jnp.full((pad,), N, jnp.int32)])
    sel = jnp.asarray(_SEL)
    selT = sel.T
    asrc = att_src.reshape(2, 1, D)
    adst = att_dst.reshape(2, 1, D)
    r1 = lambda v: v.reshape(1, D)

    xpa0, ald0 = _tc_pre(x, gat_W[0], asrc[0], adst[0], sel)
    acc0 = _sc_edge(srcp, dstp, xpa0, ald0).reshape(_NC, NROW, FW)
    x1, xpa1, ald1 = _tc_mid(
        acc0, x, h_W[0], r1(h_b[0]), res_W[0], r1(res_b[0]), r1(ln_g[0]),
        r1(ln_b[0]), r1(betas[0]), selT, gat_W[1], asrc[1], adst[1], sel)
    acc1 = _sc_edge(srcp, dstp, xpa1, ald1).reshape(_NC, NROW, FW)
    x_out = _tc_final(
        acc1, x1, h_W[1], r1(h_b[1]), res_W[1], r1(res_b[1]), r1(ln_g[1]),
        r1(ln_b[1]), r1(betas[1]), selT)
    return x_out


# speedup vs baseline: 101.0039x; 1.8945x over previous
import functools

import jax
import jax.numpy as jnp
import numpy as np
from jax import lax
from jax.experimental import pallas as pl
from jax.experimental.pallas import tpu as pltpu
from jax.experimental.pallas import tpu_sc as plsc

N = 10000
D = 128
H = 8
C = 16
E = 320000

_NC = 2
_NS = 16
_NW = _NC * _NS
CK = 80
CPT = 126
EPT = CPT * CK
E_PAD = _NW * EPT
NB = 3
NROW = N + 112
RPT = NROW // _NS
FW = 144

_SEL = np.zeros((D, H), np.float32)
_SEL[np.arange(D), np.arange(D) // C] = 1.0

_B = 1000


def _sc_edge_body(epk, xpa, ald16, out, acc_sh, msg0, msg1, msg2, ald0, ald1,
                  ald2, idx0, idx1, idx2, srv0, srv1, srv2, dsv0, dsv1, dsv2,
                  dcv0, dcv1, dcv2, semI, semG, semS):
    cid = lax.axis_index("c")
    sid = lax.axis_index("s")
    wid = cid * _NS + sid
    msg = (msg0, msg1, msg2)
    ald = (ald0, ald1, ald2)
    idx = (idx0, idx1, idx2)
    srv = (srv0, srv1, srv2)
    dsv = (dsv0, dsv1, dsv2)
    dcv = (dcv0, dcv1, dcv2)

    def _zero_msg(i, carry):
        for j in range(FW // 16):
            msg0[i, pl.ds(j * 16, 16)] = jnp.zeros((16,), jnp.float32)
        return carry

    lax.fori_loop(0, CK, _zero_msg, 0)
    zbase = sid * RPT
    nfull, zrem = RPT // CK, RPT % CK
    for j in range(nfull):
        pltpu.sync_copy(msg0, acc_sh.at[pl.ds(zbase + j * CK, CK)])
    if zrem:
        pltpu.sync_copy(msg0.at[pl.ds(0, zrem)],
                        acc_sh.at[pl.ds(zbase + nfull * CK, zrem)])
    plsc.subcore_barrier()

    def _issue_idx(c, b):
        pltpu.async_copy(epk.at[wid * CPT + c], idx[b], semI.at[b])

    def _wait_idx(b):
        pltpu.make_async_copy(epk.at[0], idx[b], semI.at[b]).wait()

    def _prep_idx(b):
        for j in range(CK // 16):
            sl = pl.ds(j * 16, 16)
            srv[b][sl] = idx[b][0, sl]
            dv = idx[b][1, sl]
            dsv[b][sl] = dv
            dcv[b][sl] = jnp.minimum(dv, N - 1)

    def _issue_gather(b):
        pltpu.async_copy(xpa.at[srv[b]], msg[b], semG.at[b])
        pltpu.async_copy(ald16.at[dcv[b]], ald[b], semG.at[b])

    def _wait_gather(b):
        pltpu.make_async_copy(xpa.at[srv[b]], msg[b], semG.at[b]).wait()
        pltpu.make_async_copy(ald16.at[dcv[b]], ald[b], semG.at[b]).wait()

    def _issue_scatter(b):
        pltpu.async_copy(msg[b], acc_sh.at[dsv[b]], semS.at[b], add=True)

    def _wait_scatter(b):
        pltpu.make_async_copy(msg[b], acc_sh.at[dsv[b]], semS.at[b]).wait()

    def _compute(b):
        mb, ab = msg[b], ald[b]

        def _edge(e, ecarry):
            al = mb[e, pl.ds(D, 16)] + ab[e, pl.ds(0, 16)]
            exv = jnp.exp(jnp.maximum(al, 0.2 * al))
            mb[e, pl.ds(D, 16)] = exv
            for h in range(H):
                w = jnp.full((16,), exv[h], jnp.float32)
                mb[e, pl.ds(h * 16, 16)] = mb[e, pl.ds(h * 16, 16)] * w
            return ecarry

        lax.fori_loop(0, CK, _edge, 0)

    _issue_idx(0, 0)
    _issue_idx(1, 1)
    _issue_idx(2, 2)
    _wait_idx(0)
    _prep_idx(0)
    _issue_gather(0)
    _wait_idx(1)
    _prep_idx(1)
    _issue_gather(1)

    def _slot(c, b, bp, bn2):
        _wait_gather(b)

        @pl.when(c + NB < CPT)
        def _():
            _issue_idx(c + NB, b)

        _compute(b)
        _issue_scatter(b)

        @pl.when(c >= 1)
        def _():
            _wait_scatter(bp)

        @pl.when(c + 2 < CPT)
        def _():
            _wait_idx(bn2)
            _prep_idx(bn2)
            _issue_gather(bn2)

    def _round(r, carry):
        for b in range(NB):
            c = r * NB + b
            _slot(c, b, (b + NB - 1) % NB, (b + 2) % NB)
        return carry

    lax.fori_loop(0, CPT // NB, _round, 0)
    _wait_scatter((CPT - 1) % NB)
    plsc.subcore_barrier()

    rbase = sid * RPT
    obase = cid * NROW + rbase
    for j in range(nfull):
        pltpu.sync_copy(acc_sh.at[pl.ds(rbase + j * CK, CK)], msg0)
        pltpu.sync_copy(msg0, out.at[pl.ds(obase + j * CK, CK)])
    if zrem:
        pltpu.sync_copy(acc_sh.at[pl.ds(rbase + nfull * CK, zrem)],
                        msg0.at[pl.ds(0, zrem)])
        pltpu.sync_copy(msg0.at[pl.ds(0, zrem)],
                        out.at[pl.ds(obase + nfull * CK, zrem)])


@functools.cache
def _make_sc_edge():
    return pl.kernel(
        _sc_edge_body,
        out_type=jax.ShapeDtypeStruct((_NC * NROW, FW), jnp.float32),
        mesh=plsc.VectorSubcoreMesh(core_axis_name="c", subcore_axis_name="s",
                                    num_cores=_NC, num_subcores=_NS),
        scratch_types=(
            [pltpu.VMEM_SHARED((NROW, FW), jnp.float32)]
            + [pltpu.VMEM((CK, FW), jnp.float32)] * NB
            + [pltpu.VMEM((CK, 16), jnp.float32)] * NB
            + [pltpu.VMEM((2, CK), jnp.int32)] * NB
            + [pltpu.VMEM((CK,), jnp.int32)] * (3 * NB)
            + [pltpu.SemaphoreType.DMA((NB,))] * 3
        ),
        compiler_params=pltpu.CompilerParams(use_tc_tiling_on_sc=False),
    )


def _sc_edge(*args):
    return _make_sc_edge()(*args)


def _pre_math(xp, asrc, adst, sel):
    als = jnp.dot(xp * asrc, sel, preferred_element_type=jnp.float32)
    ald = jnp.dot(xp * adst, sel, preferred_element_type=jnp.float32)
    z = jnp.zeros((xp.shape[0], H), jnp.float32)
    return (jnp.concatenate([xp, als, z], axis=1),
            jnp.concatenate([ald, z], axis=1))


def _tc_pre_body(x_ref, w_ref, asrc_ref, adst_ref, sel_ref, xpa_ref, ald_ref):
    xp = jnp.dot(x_ref[...], w_ref[...], preferred_element_type=jnp.float32)
    xpa, ald = _pre_math(xp, asrc_ref[...], adst_ref[...], sel_ref[...])
    xpa_ref[...] = xpa
    ald_ref[...] = ald


def _post_math(acc_ref, x, hw, hb, rw, rb, g, b, beta, selT):
    S = acc_ref[0] + acc_ref[1]
    den128 = jnp.dot(S[:, D:D + H], selT, preferred_element_type=jnp.float32)
    gat = S[:, :D] / (den128 + 1e-16)
    h = jnp.maximum(jnp.dot(x, hw, preferred_element_type=jnp.float32) + hb, 0.0)
    res = jnp.dot(x, rw, preferred_element_type=jnp.float32) + rb
    av = jnp.maximum(gat + res, 0.0)
    t = av * h
    m = jnp.mean(t, axis=-1, keepdims=True)
    var = jnp.mean((t - m) ** 2, axis=-1, keepdims=True)
    ln = (t - m) * lax.rsqrt(var + 1e-5) * g + b
    return (1.0 - beta) * ln + beta * av


def _tc_mid_body(acc_ref, x_ref, hw_ref, hb_ref, rw_ref, rb_ref, g_ref, b_ref,
                 beta_ref, selT_ref, w2_ref, asrc_ref, adst_ref, sel_ref,
                 xn_ref, xpa_ref, ald_ref):
    xn = _post_math(acc_ref, x_ref[...], hw_ref[...], hb_ref[...], rw_ref[...],
                    rb_ref[...], g_ref[...], b_ref[...], beta_ref[...],
                    selT_ref[...])
    xn_ref[...] = xn
    xp = jnp.dot(xn, w2_ref[...], preferred_element_type=jnp.float32)
    xpa, ald = _pre_math(xp, asrc_ref[...], adst_ref[...], sel_ref[...])
    xpa_ref[...] = xpa
    ald_ref[...] = ald


def _tc_final_body(acc_ref, x_ref, hw_ref, hb_ref, rw_ref, rb_ref, g_ref,
                   b_ref, beta_ref, selT_ref, out_ref):
    xn = _post_math(acc_ref, x_ref[...], hw_ref[...], hb_ref[...], rw_ref[...],
                    rb_ref[...], g_ref[...], b_ref[...], beta_ref[...],
                    selT_ref[...])
    out_ref[...] = xn + x_ref[...]


def _row_spec(w):
    return pl.BlockSpec((_B, w), lambda i: (i, 0))


def _full_spec(r, c):
    return pl.BlockSpec((r, c), lambda i: (0, 0))


_ACC_SPEC = pl.BlockSpec((_NC, _B, FW), lambda i: (0, i, 0))

_tc_pre = pl.pallas_call(
    _tc_pre_body,
    grid=(N // _B,),
    in_specs=[_row_spec(D), _full_spec(D, D), _full_spec(1, D),
              _full_spec(1, D), _full_spec(D, H)],
    out_specs=[_row_spec(FW), _row_spec(16)],
    out_shape=[jax.ShapeDtypeStruct((N, FW), jnp.float32),
               jax.ShapeDtypeStruct((N, 16), jnp.float32)],
)

_POST_SPECS = [_ACC_SPEC, _row_spec(D), _full_spec(D, D), _full_spec(1, D),
               _full_spec(D, D), _full_spec(1, D), _full_spec(1, D),
               _full_spec(1, D), _full_spec(1, D), _full_spec(H, D)]

_tc_mid = pl.pallas_call(
    _tc_mid_body,
    grid=(N // _B,),
    in_specs=_POST_SPECS + [_full_spec(D, D), _full_spec(1, D),
                            _full_spec(1, D), _full_spec(D, H)],
    out_specs=[_row_spec(D), _row_spec(FW), _row_spec(16)],
    out_shape=[jax.ShapeDtypeStruct((N, D), jnp.float32),
               jax.ShapeDtypeStruct((N, FW), jnp.float32),
               jax.ShapeDtypeStruct((N, 16), jnp.float32)],
)

_tc_final = pl.pallas_call(
    _tc_final_body,
    grid=(N // _B,),
    in_specs=_POST_SPECS,
    out_specs=_row_spec(D),
    out_shape=jax.ShapeDtypeStruct((N, D), jnp.float32),
)


def kernel(x, edge_index, h_W, h_b, gat_W, att_src, att_dst, res_W, res_b,
           ln_g, ln_b, betas):
    pad = E_PAD - E
    srcp = jnp.concatenate([edge_index[0], jnp.zeros((pad,), jnp.int32)])
    dstp = jnp.concatenate([edge_index[1], jnp.full((pad,), N, jnp.int32)])
    epk = jnp.stack([srcp.reshape(-1, CK), dstp.reshape(-1, CK)], axis=1)
    sel = jnp.asarray(_SEL)
    selT = sel.T
    asrc = att_src.reshape(2, 1, D)
    adst = att_dst.reshape(2, 1, D)
    r1 = lambda v: v.reshape(1, D)

    xpa0, ald0 = _tc_pre(x, gat_W[0], asrc[0], adst[0], sel)
    acc0 = _sc_edge(epk, xpa0, ald0).reshape(_NC, NROW, FW)
    x1, xpa1, ald1 = _tc_mid(
        acc0, x, h_W[0], r1(h_b[0]), res_W[0], r1(res_b[0]), r1(ln_g[0]),
        r1(ln_b[0]), r1(betas[0]), selT, gat_W[1], asrc[1], adst[1], sel)
    acc1 = _sc_edge(epk, xpa1, ald1).reshape(_NC, NROW, FW)
    x_out = _tc_final(
        acc1, x1, h_W[1], r1(h_b[1]), res_W[1], r1(res_b[1]), r1(ln_g[1]),
        r1(ln_b[1]), r1(betas[1]), selT)
    return x_out
